# baseline (speedup 1.0000x reference)
import functools

import jax
import jax.numpy as jnp
from jax import lax
from jax.experimental import pallas as pl
from jax.experimental.pallas import tpu as pltpu
from jax.experimental.pallas import tpu_sc as plsc

_N = 10000
_E = 320000
_G = 64
_NC = 2
_NS = 16
_K = 64
_EP = 327680
_NPT = 632
_ND = _NS * _NPT
_PAD_AE = -3e38


def _tc_edge_body(ea_ref, We1_ref, ae1_ref, We2_ref, ae2_ref,
                  ae1o_ref, ae2o_ref, mxe_ref):
    g = pl.program_id(0)
    ea = ea_ref[...]
    ii = lax.broadcasted_iota(jnp.int32, (128, 128), 0)
    cc = lax.broadcasted_iota(jnp.int32, (128, 128), 1)
    mask = (ii // 16) == (cc % 8)

    def logits(We_ref, ae_ref):
        ve = jnp.dot(We_ref[...], ae_ref[...],
                     preferred_element_type=jnp.float32)
        rep = jnp.concatenate([ve] * 8, axis=0)
        M = jnp.where(mask, rep, 0.0)
        out = jnp.dot(ea, M, preferred_element_type=jnp.float32)
        return out[:, :8]

    o1 = logits(We1_ref, ae1_ref)
    o2 = logits(We2_ref, ae2_ref)
    ae1o_ref[...] = o1
    ae2o_ref[...] = o2
    lane = lax.broadcasted_iota(jnp.int32, (1, 128), 1)
    cur = jnp.where(lane == 0, jnp.max(o1),
                    jnp.where(lane == 1, jnp.max(o2), -3e38))

    @pl.when(g == 0)
    def _():
        mxe_ref[...] = jnp.full((1, 128), -3e38, jnp.float32)

    mxe_ref[...] = jnp.maximum(mxe_ref[...], cur)


_EB = 4000


def _tc_edge(ea2, We1, ae1c, We2, ae2c):
    return pl.pallas_call(
        _tc_edge_body,
        grid=(_E // 8 // _EB,),
        in_specs=[
            pl.BlockSpec((_EB, 128), lambda g: (g, 0)),
            pl.BlockSpec((16, 128), lambda g: (0, 0)),
            pl.BlockSpec((128, 1), lambda g: (0, 0)),
            pl.BlockSpec((16, 256), lambda g: (0, 0)),
            pl.BlockSpec((256, 1), lambda g: (0, 0)),
        ],
        out_specs=[
            pl.BlockSpec((_EB, 8), lambda g: (g, 0)),
            pl.BlockSpec((_EB, 8), lambda g: (g, 0)),
            pl.BlockSpec((1, 128), lambda g: (0, 0)),
        ],
        out_shape=[
            jax.ShapeDtypeStruct((_E // 8, 8), jnp.float32),
            jax.ShapeDtypeStruct((_E // 8, 8), jnp.float32),
            jax.ShapeDtypeStruct((1, 128), jnp.float32),
        ],
    )(ea2, We1, ae1c, We2, ae2c)


def _tc_pre_body(x_ref, W1_ref, ad_ref, mxe_ref,
                 xps_ref, asd_ref, cvec_ref):
    xp = jnp.dot(x_ref[...], W1_ref[...], preferred_element_type=jnp.float32)
    xps_ref[...] = xp.astype(jnp.bfloat16)
    asd = jnp.dot(xp, ad_ref[...], preferred_element_type=jnp.float32)
    asd_ref[...] = asd
    b = jnp.max(asd[:, 0:1]) + jnp.max(asd[:, 1:2]) + mxe_ref[0, 0]
    c = jnp.where(b >= 0.0, b, 0.2 * b)
    cvec_ref[...] = jnp.full((1, 128), c, jnp.float32)


def _tc_pre(x, W1, ad_pack, mxe):
    return pl.pallas_call(
        _tc_pre_body,
        out_shape=[
            jax.ShapeDtypeStruct((_N, 128), jnp.bfloat16),
            jax.ShapeDtypeStruct((_N, 2), jnp.float32),
            jax.ShapeDtypeStruct((1, 128), jnp.float32),
        ],
    )(x, W1, ad_pack, mxe)


def _tc_mid_body(agg_ref, den_ref, b1_ref, W2_ref, ad_ref, mxe_ref,
                 xps_ref, asd_ref, cvec_ref):
    agg = agg_ref[0:_N, :] + agg_ref[_ND:_ND + _N, :]
    den = den_ref[0:_N, :] + den_ref[_ND:_ND + _N, :] + 1e-16
    h = jnp.maximum(agg / den + b1_ref[...], 0.0)
    xp = jnp.dot(h, W2_ref[...], preferred_element_type=jnp.float32)
    xpb = xp.astype(jnp.bfloat16)
    xps_ref[0:_N, :] = xpb[:, :128]
    xps_ref[_N:2 * _N, :] = xpb[:, 128:]
    asd = jnp.dot(xp, ad_ref[...], preferred_element_type=jnp.float32)
    asd_ref[...] = asd
    b = jnp.max(asd[:, 0:1]) + jnp.max(asd[:, 1:2]) + mxe_ref[0, 1]
    c = jnp.where(b >= 0.0, b, 0.2 * b)
    cvec_ref[...] = jnp.full((1, 128), c, jnp.float32)


def _tc_mid(agg1, den1, b1r, W2, ad_pack, mxe):
    return pl.pallas_call(
        _tc_mid_body,
        out_shape=[
            jax.ShapeDtypeStruct((2 * _N, 128), jnp.bfloat16),
            jax.ShapeDtypeStruct((_N, 2), jnp.float32),
            jax.ShapeDtypeStruct((1, 128), jnp.float32),
        ],
    )(agg1, den1, b1r, W2, ad_pack, mxe)


def _tc_post_body(agg_ref, den_ref, b2_ref, batch_ref, Wfc_ref, bfc_ref, out_ref):
    agg = jnp.concatenate([agg_ref[0:_N, :], agg_ref[_ND:_ND + _N, :]], axis=1)
    den = den_ref[0:_N, :] + 1e-16
    h = jnp.maximum(agg / den + b2_ref[...], 0.0)
    gi = lax.broadcasted_iota(jnp.int32, (_G, 1), 0)
    P = (batch_ref[0:1, :] == gi).astype(jnp.float32)
    pooled = jnp.dot(P, h, preferred_element_type=jnp.float32)
    z = jnp.dot(pooled, Wfc_ref[...], preferred_element_type=jnp.float32)
    z = z + bfc_ref[...]
    out_ref[...] = 1.0 / (1.0 + jnp.exp(-z))


def _tc_post(agg2, den2, b2r, batch2d, Wfc, bfcr):
    return pl.pallas_call(
        _tc_post_body,
        out_shape=jax.ShapeDtypeStruct((_G, 1), jnp.float32),
    )(agg2, den2, b2r, batch2d, Wfc, bfcr)


def _sc_edge_body(mode, asd_h, ae_h, src_h, dst_h, cv_h, xp_h,
                  agg_o, den_o,
                  asd_v, cvec_v, src_v, dst_v, gidx_v, sidx_v, ae_v, ex_v,
                  rowsb_v, rows_v, zr_v, zd_v, accum, denom,
                  sem_in0, sem_in1, sem_g0, sem_g1, sem_s0, sem_s1):
    dh = 128
    cid = lax.axis_index("c")
    sid = lax.axis_index("s")
    sem_in = (sem_in0, sem_in1)
    sem_g = (sem_g0, sem_g1)
    sem_s = (sem_s0, sem_s1)
    n_chunks = (_EP // _NC // _NS // _K) if mode == 0 else (_EP // _NS // _K)

    def ebase(t):
        if mode == 0:
            return cid * (_EP // _NC) + sid * (_EP // _NC // _NS) + t * _K
        return sid * (_EP // _NS) + t * _K

    def stage_asd(q, carry):
        sl = pl.ds(q * 800, 800)
        pltpu.sync_copy(asd_h.at[sl], asd_v.at[sl])
        return carry

    lax.fori_loop(0, (2 * _N) // 800, stage_asd, 0)
    pltpu.sync_copy(cv_h, cvec_v)
    cv = cvec_v[...]
    zvec = jnp.zeros((16,), jnp.float32)
    for r in range(8):
        for j in range(dh // 16):
            zr_v[r, pl.ds(j * 16, 16)] = zvec
    for j in range(640 // 16):
        zd_v[pl.ds(j * 16, 16)] = zvec
    r0 = sid * _NPT

    def zfill(q, carry):
        pltpu.sync_copy(zr_v, accum.at[pl.ds(r0 + q * 8, 8), :])
        return carry

    lax.fori_loop(0, _NPT // 8, zfill, 0)
    pltpu.sync_copy(zd_v.at[pl.ds(0, _NPT)], denom.at[pl.ds(r0, _NPT)])
    plsc.subcore_barrier()

    cN = cid * _N
    zi = jnp.zeros((16,), jnp.int32)

    def issue_in(t, b):
        base = ebase(t)
        pltpu.async_copy(src_h.at[pl.ds(base, _K)], src_v.at[b], sem_in[b])
        pltpu.async_copy(dst_h.at[pl.ds(base, _K)], dst_v.at[b], sem_in[b])
        pltpu.async_copy(ae_h.at[pl.ds(base, _K)], ae_v.at[b], sem_in[b])

    def wait_in(t, b):
        base = ebase(t)
        pltpu.make_async_copy(src_h.at[pl.ds(base, _K)], src_v.at[b], sem_in[b]).wait()
        pltpu.make_async_copy(dst_h.at[pl.ds(base, _K)], dst_v.at[b], sem_in[b]).wait()
        pltpu.make_async_copy(ae_h.at[pl.ds(base, _K)], ae_v.at[b], sem_in[b]).wait()

    def compute_ex(b):
        for i in range(_K // 16):
            sl = pl.ds(i * 16, 16)
            s16 = src_v[b, sl]
            d16 = dst_v[b, sl]
            asg = plsc.load_gather(asd_v, [s16 * 2])
            adg = plsc.load_gather(asd_v, [d16 * 2 + 1])
            ss = asg + adg + ae_v[b, sl]
            al = jnp.where(ss >= 0.0, ss, ss * 0.2)
            ex_v[b, sl] = jnp.exp(al - cv)
            gidx_v[b, sl] = s16 if mode == 0 else s16 + cN
            sidx_v[b, sl] = d16

    def issue_gather(b):
        pltpu.async_copy(xp_h.at[gidx_v.at[b]], rowsb_v.at[b], sem_g[b])

    def wait_gather(b):
        pltpu.make_async_copy(xp_h.at[gidx_v.at[b]], rowsb_v.at[b], sem_g[b]).wait()

    def scale_rows(b):
        bb = jnp.full((16,), b, jnp.int32)

        def se(g, c2):
            e0 = g * 16
            for l in range(16):
                e = e0 + l
                sc = plsc.load_gather(ex_v, [bb, jnp.full((16,), e, jnp.int32)])
                for j in range(dh // 32):
                    v = rowsb_v[b, e, pl.ds(j * 32, 32)]
                    lo, hi = plsc.unpack(v, format=plsc.PackFormat.INTERLEAVED)
                    rows_v[b, e, pl.ds(j * 32, 16)] = lo * sc
                    rows_v[b, e, pl.ds(j * 32 + 16, 16)] = hi * sc
            return c2

        lax.fori_loop(0, _K // 16, se, 0)

    def issue_scat(b):
        pltpu.async_copy(ex_v.at[b], denom.at[sidx_v.at[b]], sem_s[b], add=True)
        pltpu.async_copy(rows_v.at[b], accum.at[sidx_v.at[b]], sem_s[b], add=True)

    def wait_scat(b):
        pltpu.make_async_copy(ex_v.at[b], denom.at[sidx_v.at[b]], sem_s[b]).wait()
        pltpu.make_async_copy(rows_v.at[b], accum.at[sidx_v.at[b]], sem_s[b]).wait()

    issue_in(0, 0)
    issue_in(1, 1)
    wait_in(0, 0)
    compute_ex(0)
    issue_gather(0)

    def pair(q, carry):
        for db in range(2):
            t = 2 * q + db
            b = db
            wait_gather(b)
            scale_rows(b)
            issue_scat(b)

            @pl.when(t >= 1)
            def _():
                wait_scat(1 - b)

            @pl.when(t + 1 < n_chunks)
            def _():
                wait_in(t + 1, 1 - b)
                compute_ex(1 - b)
                issue_gather(1 - b)

                @pl.when(t + 2 < n_chunks)
                def _():
                    issue_in(t + 2, b)
        return carry

    lax.fori_loop(0, n_chunks // 2, pair, 0)
    wait_scat(1)
    plsc.subcore_barrier()
    pltpu.sync_copy(accum.at[pl.ds(r0, _NPT), :],
                    agg_o.at[pl.ds(cid * _ND + r0, _NPT), :])
    pltpu.sync_copy(denom.at[pl.ds(r0, _NPT)], zd_v.at[pl.ds(0, _NPT)])
    pltpu.sync_copy(zd_v.at[pl.ds(0, _NPT)], den_o.at[cid, pl.ds(r0, _NPT)])


def _make_sc_edge(mode):
    dh = 128
    mesh = plsc.VectorSubcoreMesh(core_axis_name="c", subcore_axis_name="s",
                                  num_cores=_NC, num_subcores=_NS)
    return pl.kernel(
        functools.partial(_sc_edge_body, mode),
        out_type=[
            jax.ShapeDtypeStruct((2 * _ND, dh), jnp.float32),
            jax.ShapeDtypeStruct((_NC, _ND), jnp.float32),
        ],
        mesh=mesh,
        compiler_params=pltpu.CompilerParams(needs_layout_passes=False, use_tc_tiling_on_sc=False),
        scratch_types=[
            pltpu.VMEM((2 * _N,), jnp.float32),
            pltpu.VMEM((16,), jnp.float32),
            pltpu.VMEM((2, _K), jnp.int32),
            pltpu.VMEM((2, _K), jnp.int32),
            pltpu.VMEM((2, _K), jnp.int32),
            pltpu.VMEM((2, _K), jnp.int32),
            pltpu.VMEM((2, _K), jnp.float32),
            pltpu.VMEM((2, _K), jnp.float32),
            pltpu.VMEM((2, _K, dh), jnp.bfloat16),
            pltpu.VMEM((2, _K, dh), jnp.float32),
            pltpu.VMEM((8, dh), jnp.float32),
            pltpu.VMEM((640,), jnp.float32),
            pltpu.VMEM_SHARED((_ND, dh), jnp.float32),
            pltpu.VMEM_SHARED((_ND,), jnp.float32),
            pltpu.SemaphoreType.DMA,
            pltpu.SemaphoreType.DMA,
            pltpu.SemaphoreType.DMA,
            pltpu.SemaphoreType.DMA,
            pltpu.SemaphoreType.DMA,
            pltpu.SemaphoreType.DMA,
        ],
        name=f"sc_gat_edge_m{mode}",
    )


_sc_edge_l1 = _make_sc_edge(0)
_sc_edge_l2 = _make_sc_edge(1)


def kernel(x, edge_index, edge_attr, batch,
           W1, We1, as1, ad1, ae1, b1,
           W2, We2, as2, ad2, ae2, b2,
           Wfc, bfc):
    src = edge_index[0].astype(jnp.int32)
    dst = edge_index[1].astype(jnp.int32)
    ea2 = edge_attr.reshape(_E // 8, 128)
    ae1o, ae2o, mxe = _tc_edge(ea2, We1, ae1.reshape(128, 1),
                               We2, ae2.reshape(256, 1))
    ae1f = ae1o.reshape(_E)
    ae2f = ae2o.reshape(_E)

    import numpy as _np
    p128 = _np.arange(128)
    f128 = 32 * (p128 // 32) + (p128 % 2) * 16 + (p128 % 32) // 2
    f256 = _np.concatenate([f128, 128 + f128])
    W1p = W1[:, f128]
    as1p = as1[f128]
    ad1p_ = ad1[f128]
    ad1p = jnp.stack([as1p, ad1p_], axis=1)
    xp1s, asd1, cvec1 = _tc_pre(x, W1p, ad1p, mxe)
    padi = jnp.zeros((_EP - _E,), jnp.int32)
    srcp = jnp.concatenate([src, padi])
    dstp = jnp.concatenate([dst, padi])
    ae1p = jnp.concatenate([ae1f, jnp.full((_EP - _E,), _PAD_AE, jnp.float32)])
    agg1, den1 = _sc_edge_l1(asd1.reshape(2 * _N), ae1p, srcp, dstp,
                             cvec1[0, :16], xp1s)

    W2p = W2[:, f256]
    ad2p = jnp.stack([as2[f256], ad2[f256]], axis=1)
    xp2s, asd2, cvec2 = _tc_mid(agg1, den1.reshape(2 * _ND, 1),
                                b1.reshape(1, 128), W2p, ad2p, mxe)
    ae2p = jnp.concatenate([ae2f, jnp.full((_EP - _E,), _PAD_AE, jnp.float32)])
    agg2, den2 = _sc_edge_l2(asd2.reshape(2 * _N), ae2p, srcp, dstp,
                             cvec2[0, :16], xp2s)

    batch2d = jnp.broadcast_to(batch.astype(jnp.int32)[None, :], (8, _N))
    out = _tc_post(agg2, den2.reshape(2 * _ND, 1), b2.reshape(1, 256),
                   batch2d, Wfc, bfc.reshape(1, 1))
    return out


# speedup vs baseline: 1.7476x; 1.7476x over previous
import functools

import jax
import jax.numpy as jnp
from jax import lax
from jax.experimental import pallas as pl
from jax.experimental.pallas import tpu as pltpu
from jax.experimental.pallas import tpu_sc as plsc

_N = 10000
_E = 320000
_G = 64
_NC = 2
_NS = 16
_K = 96
_EP = 325632
_NPT = 632
_ND = _NS * _NPT
_PAD_AE = -3e38


def _tc_edge_body(ea_ref, We1_ref, ae1_ref, We2_ref, ae2_ref,
                  ae1o_ref, ae2o_ref, mxe_ref):
    g = pl.program_id(0)
    ea = ea_ref[...]
    ii = lax.broadcasted_iota(jnp.int32, (128, 128), 0)
    cc = lax.broadcasted_iota(jnp.int32, (128, 128), 1)
    mask = (ii // 16) == (cc % 8)

    def logits(We_ref, ae_ref):
        ve = jnp.dot(We_ref[...], ae_ref[...],
                     preferred_element_type=jnp.float32)
        rep = jnp.concatenate([ve] * 8, axis=0)
        M = jnp.where(mask, rep, 0.0)
        out = jnp.dot(ea, M, preferred_element_type=jnp.float32)
        return out[:, :8]

    o1 = logits(We1_ref, ae1_ref)
    o2 = logits(We2_ref, ae2_ref)
    ae1o_ref[...] = o1
    ae2o_ref[...] = o2
    lane = lax.broadcasted_iota(jnp.int32, (1, 128), 1)
    cur = jnp.where(lane == 0, jnp.max(o1),
                    jnp.where(lane == 1, jnp.max(o2), -3e38))

    @pl.when(g == 0)
    def _():
        mxe_ref[...] = jnp.full((1, 128), -3e38, jnp.float32)

    mxe_ref[...] = jnp.maximum(mxe_ref[...], cur)


_EB = 4000


def _tc_edge(ea2, We1, ae1c, We2, ae2c):
    return pl.pallas_call(
        _tc_edge_body,
        grid=(_E // 8 // _EB,),
        in_specs=[
            pl.BlockSpec((_EB, 128), lambda g: (g, 0)),
            pl.BlockSpec((16, 128), lambda g: (0, 0)),
            pl.BlockSpec((128, 1), lambda g: (0, 0)),
            pl.BlockSpec((16, 256), lambda g: (0, 0)),
            pl.BlockSpec((256, 1), lambda g: (0, 0)),
        ],
        out_specs=[
            pl.BlockSpec((_EB, 8), lambda g: (g, 0)),
            pl.BlockSpec((_EB, 8), lambda g: (g, 0)),
            pl.BlockSpec((1, 128), lambda g: (0, 0)),
        ],
        out_shape=[
            jax.ShapeDtypeStruct((_E // 8, 8), jnp.float32),
            jax.ShapeDtypeStruct((_E // 8, 8), jnp.float32),
            jax.ShapeDtypeStruct((1, 128), jnp.float32),
        ],
    )(ea2, We1, ae1c, We2, ae2c)


def _tc_pre_body(x_ref, W1_ref, ad_ref, mxe_ref,
                 xps_ref, asd_ref, cvec_ref):
    xp = jnp.dot(x_ref[...], W1_ref[...], preferred_element_type=jnp.float32)
    xps_ref[...] = xp
    asd = jnp.dot(xp, ad_ref[...], preferred_element_type=jnp.float32)
    asd_ref[...] = asd
    b = jnp.max(asd[:, 0:1]) + jnp.max(asd[:, 1:2]) + mxe_ref[0, 0]
    c = jnp.where(b >= 0.0, b, 0.2 * b)
    cvec_ref[...] = jnp.full((1, 128), c, jnp.float32)


def _tc_pre(x, W1, ad_pack, mxe):
    return pl.pallas_call(
        _tc_pre_body,
        out_shape=[
            jax.ShapeDtypeStruct((_N, 128), jnp.float32),
            jax.ShapeDtypeStruct((_N, 2), jnp.float32),
            jax.ShapeDtypeStruct((1, 128), jnp.float32),
        ],
    )(x, W1, ad_pack, mxe)


def _tc_mid_body(agg_ref, den_ref, b1_ref, W2_ref, ad_ref, mxe_ref,
                 xps_ref, asd_ref, cvec_ref):
    agg = agg_ref[0:_N, :] + agg_ref[_ND:_ND + _N, :]
    den = den_ref[0:_N, :] + den_ref[_ND:_ND + _N, :] + 1e-16
    h = jnp.maximum(agg / den + b1_ref[...], 0.0)
    xp = jnp.dot(h, W2_ref[...], preferred_element_type=jnp.float32)
    xps_ref[0:_N, :] = xp[:, :128]
    xps_ref[_N:2 * _N, :] = xp[:, 128:]
    asd = jnp.dot(xp, ad_ref[...], preferred_element_type=jnp.float32)
    asd_ref[...] = asd
    b = jnp.max(asd[:, 0:1]) + jnp.max(asd[:, 1:2]) + mxe_ref[0, 1]
    c = jnp.where(b >= 0.0, b, 0.2 * b)
    cvec_ref[...] = jnp.full((1, 128), c, jnp.float32)


def _tc_mid(agg1, den1, b1r, W2, ad_pack, mxe):
    return pl.pallas_call(
        _tc_mid_body,
        out_shape=[
            jax.ShapeDtypeStruct((2 * _N, 128), jnp.float32),
            jax.ShapeDtypeStruct((_N, 2), jnp.float32),
            jax.ShapeDtypeStruct((1, 128), jnp.float32),
        ],
    )(agg1, den1, b1r, W2, ad_pack, mxe)


def _tc_post_body(agg_ref, den_ref, b2_ref, batch_ref, Wfc_ref, bfc_ref, out_ref):
    agg = jnp.concatenate([agg_ref[0:_N, :], agg_ref[_ND:_ND + _N, :]], axis=1)
    den = den_ref[0:_N, :] + 1e-16
    h = jnp.maximum(agg / den + b2_ref[...], 0.0)
    gi = lax.broadcasted_iota(jnp.int32, (_G, 1), 0)
    P = (batch_ref[0:1, :] == gi).astype(jnp.float32)
    pooled = jnp.dot(P, h, preferred_element_type=jnp.float32)
    z = jnp.dot(pooled, Wfc_ref[...], preferred_element_type=jnp.float32)
    z = z + bfc_ref[...]
    out_ref[...] = 1.0 / (1.0 + jnp.exp(-z))


def _tc_post(agg2, den2, b2r, batch2d, Wfc, bfcr):
    return pl.pallas_call(
        _tc_post_body,
        out_shape=jax.ShapeDtypeStruct((_G, 1), jnp.float32),
    )(agg2, den2, b2r, batch2d, Wfc, bfcr)


def _sc_edge_body(mode, asd_h, ae_h, src_h, dst_h, cv_h, xp_h,
                  agg_o, den_o,
                  asd_v, cvec_v, src_v, dst_v, gidx_v, sidx_v, ae_v, ex_v,
                  rows_v, zr_v, zd_v, accum, denom,
                  sem_in0, sem_in1, sem_g0, sem_g1, sem_s0, sem_s1):
    dh = 128
    cid = lax.axis_index("c")
    sid = lax.axis_index("s")
    sem_in = (sem_in0, sem_in1)
    sem_g = (sem_g0, sem_g1)
    sem_s = (sem_s0, sem_s1)
    n_chunks = (_EP // _NC // _NS // _K) if mode == 0 else (_EP // _NS // _K)

    def ebase(t):
        if mode == 0:
            return cid * (_EP // _NC) + sid * (_EP // _NC // _NS) + t * _K
        return sid * (_EP // _NS) + t * _K

    def stage_asd(q, carry):
        sl = pl.ds(q * 800, 800)
        pltpu.sync_copy(asd_h.at[sl], asd_v.at[sl])
        return carry

    lax.fori_loop(0, (2 * _N) // 800, stage_asd, 0)
    pltpu.sync_copy(cv_h, cvec_v)
    cv = cvec_v[...]
    zvec = jnp.zeros((16,), jnp.float32)
    for r in range(8):
        for j in range(dh // 16):
            zr_v[r, pl.ds(j * 16, 16)] = zvec
    for j in range(640 // 16):
        zd_v[pl.ds(j * 16, 16)] = zvec
    r0 = sid * _NPT

    def zfill(q, carry):
        pltpu.sync_copy(zr_v, accum.at[pl.ds(r0 + q * 8, 8), :])
        return carry

    lax.fori_loop(0, _NPT // 8, zfill, 0)
    pltpu.sync_copy(zd_v.at[pl.ds(0, _NPT)], denom.at[pl.ds(r0, _NPT)])
    plsc.subcore_barrier()

    cN = cid * _N
    zi = jnp.zeros((16,), jnp.int32)

    def issue_in(t, b):
        base = ebase(t)
        pltpu.async_copy(src_h.at[pl.ds(base, _K)], src_v.at[b], sem_in[b])
        pltpu.async_copy(dst_h.at[pl.ds(base, _K)], dst_v.at[b], sem_in[b])
        pltpu.async_copy(ae_h.at[pl.ds(base, _K)], ae_v.at[b], sem_in[b])

    def wait_in(t, b):
        base = ebase(t)
        pltpu.make_async_copy(src_h.at[pl.ds(base, _K)], src_v.at[b], sem_in[b]).wait()
        pltpu.make_async_copy(dst_h.at[pl.ds(base, _K)], dst_v.at[b], sem_in[b]).wait()
        pltpu.make_async_copy(ae_h.at[pl.ds(base, _K)], ae_v.at[b], sem_in[b]).wait()

    def compute_ex(b):
        for i in range(_K // 16):
            sl = pl.ds(i * 16, 16)
            s16 = src_v[b, sl]
            d16 = dst_v[b, sl]
            asg = plsc.load_gather(asd_v, [s16 * 2])
            adg = plsc.load_gather(asd_v, [d16 * 2 + 1])
            ss = asg + adg + ae_v[b, sl]
            al = jnp.where(ss >= 0.0, ss, ss * 0.2)
            ex_v[b, sl] = jnp.exp(al - cv)
            gidx_v[b, sl] = s16 if mode == 0 else s16 + cN
            sidx_v[b, sl] = d16

    _H = _K // 2

    def issue_gather(b):
        pltpu.async_copy(xp_h.at[gidx_v.at[b, pl.ds(0, _H)]],
                         rows_v.at[b, pl.ds(0, _H), :], sem_g[b])
        pltpu.async_copy(xp_h.at[gidx_v.at[b, pl.ds(_H, _H)]],
                         rows_v.at[b, pl.ds(_H, _H), :], sem_g[b])

    def wait_gather(b):
        pltpu.make_async_copy(xp_h.at[gidx_v.at[b, pl.ds(0, _H)]],
                              rows_v.at[b, pl.ds(0, _H), :], sem_g[b]).wait()
        pltpu.make_async_copy(xp_h.at[gidx_v.at[b, pl.ds(_H, _H)]],
                              rows_v.at[b, pl.ds(_H, _H), :], sem_g[b]).wait()

    def scale_rows(b):
        bb = jnp.full((16,), b, jnp.int32)

        def se(g, c2):
            e0 = g * 16
            for l in range(16):
                e = e0 + l
                sc = plsc.load_gather(ex_v, [bb, jnp.full((16,), e, jnp.int32)])
                for j in range(dh // 16):
                    slj = pl.ds(j * 16, 16)
                    rows_v[b, e, slj] = rows_v[b, e, slj] * sc
            return c2

        lax.fori_loop(0, _K // 16, se, 0)

    def issue_scat(b):
        pltpu.async_copy(ex_v.at[b], denom.at[sidx_v.at[b]], sem_s[b], add=True)
        pltpu.async_copy(rows_v.at[b], accum.at[sidx_v.at[b]], sem_s[b], add=True)

    def wait_scat(b):
        pltpu.make_async_copy(ex_v.at[b], denom.at[sidx_v.at[b]], sem_s[b]).wait()
        pltpu.make_async_copy(rows_v.at[b], accum.at[sidx_v.at[b]], sem_s[b]).wait()

    issue_in(0, 0)
    issue_in(1, 1)
    wait_in(0, 0)
    compute_ex(0)
    issue_gather(0)

    def pair(q, carry):
        for db in range(2):
            t = 2 * q + db
            b = db
            wait_gather(b)
            scale_rows(b)
            issue_scat(b)

            @pl.when(t >= 1)
            def _():
                wait_scat(1 - b)

            @pl.when(t + 1 < n_chunks)
            def _():
                wait_in(t + 1, 1 - b)
                compute_ex(1 - b)
                issue_gather(1 - b)

                @pl.when(t + 2 < n_chunks)
                def _():
                    issue_in(t + 2, b)
        return carry

    lax.fori_loop(0, n_chunks // 2, pair, 0)
    wait_scat(1)
    plsc.subcore_barrier()
    pltpu.sync_copy(accum.at[pl.ds(r0, _NPT), :],
                    agg_o.at[pl.ds(cid * _ND + r0, _NPT), :])
    pltpu.sync_copy(denom.at[pl.ds(r0, _NPT)], zd_v.at[pl.ds(0, _NPT)])
    pltpu.sync_copy(zd_v.at[pl.ds(0, _NPT)], den_o.at[cid, pl.ds(r0, _NPT)])


def _make_sc_edge(mode):
    dh = 128
    mesh = plsc.VectorSubcoreMesh(core_axis_name="c", subcore_axis_name="s",
                                  num_cores=_NC, num_subcores=_NS)
    return pl.kernel(
        functools.partial(_sc_edge_body, mode),
        out_type=[
            jax.ShapeDtypeStruct((2 * _ND, dh), jnp.float32),
            jax.ShapeDtypeStruct((_NC, _ND), jnp.float32),
        ],
        mesh=mesh,
        compiler_params=pltpu.CompilerParams(needs_layout_passes=False, use_tc_tiling_on_sc=False),
        scratch_types=[
            pltpu.VMEM((2 * _N,), jnp.float32),
            pltpu.VMEM((16,), jnp.float32),
            pltpu.VMEM((2, _K), jnp.int32),
            pltpu.VMEM((2, _K), jnp.int32),
            pltpu.VMEM((2, _K), jnp.int32),
            pltpu.VMEM((2, _K), jnp.int32),
            pltpu.VMEM((2, _K), jnp.float32),
            pltpu.VMEM((2, _K), jnp.float32),
            pltpu.VMEM((2, _K, dh), jnp.float32),
            pltpu.VMEM((8, dh), jnp.float32),
            pltpu.VMEM((640,), jnp.float32),
            pltpu.VMEM_SHARED((_ND, dh), jnp.float32),
            pltpu.VMEM_SHARED((_ND,), jnp.float32),
            pltpu.SemaphoreType.DMA,
            pltpu.SemaphoreType.DMA,
            pltpu.SemaphoreType.DMA,
            pltpu.SemaphoreType.DMA,
            pltpu.SemaphoreType.DMA,
            pltpu.SemaphoreType.DMA,
        ],
        name=f"sc_gat_edge_m{mode}",
    )


_sc_edge_l1 = _make_sc_edge(0)
_sc_edge_l2 = _make_sc_edge(1)


def kernel(x, edge_index, edge_attr, batch,
           W1, We1, as1, ad1, ae1, b1,
           W2, We2, as2, ad2, ae2, b2,
           Wfc, bfc):
    src = edge_index[0].astype(jnp.int32)
    dst = edge_index[1].astype(jnp.int32)
    ea2 = edge_attr.reshape(_E // 8, 128)
    ae1o, ae2o, mxe = _tc_edge(ea2, We1, ae1.reshape(128, 1),
                               We2, ae2.reshape(256, 1))
    ae1f = ae1o.reshape(_E)
    ae2f = ae2o.reshape(_E)

    ad1p = jnp.stack([as1, ad1], axis=1)
    xp1s, asd1, cvec1 = _tc_pre(x, W1, ad1p, mxe)
    padi = (jnp.arange(_EP - _E, dtype=jnp.int32) * 7) % _N
    srcp = jnp.concatenate([src, padi])
    dstp = jnp.concatenate([dst, padi])
    ae1p = jnp.concatenate([ae1f, jnp.full((_EP - _E,), _PAD_AE, jnp.float32)])
    agg1, den1 = _sc_edge_l1(asd1.reshape(2 * _N), ae1p, srcp, dstp,
                             cvec1[0, :16], xp1s)

    ad2p = jnp.stack([as2, ad2], axis=1)
    xp2s, asd2, cvec2 = _tc_mid(agg1, den1.reshape(2 * _ND, 1),
                                b1.reshape(1, 128), W2, ad2p, mxe)
    ae2p = jnp.concatenate([ae2f, jnp.full((_EP - _E,), _PAD_AE, jnp.float32)])
    agg2, den2 = _sc_edge_l2(asd2.reshape(2 * _N), ae2p, srcp, dstp,
                             cvec2[0, :16], xp2s)

    batch2d = jnp.broadcast_to(batch.astype(jnp.int32)[None, :], (8, _N))
    out = _tc_post(agg2, den2.reshape(2 * _ND, 1), b2.reshape(1, 256),
                   batch2d, Wfc, bfc.reshape(1, 1))
    return out
